# baseline (speedup 1.0000x reference)
import jax
import jax.numpy as jnp
from jax import lax
from jax.experimental import pallas as pl
from jax.experimental.pallas import tpu as pltpu
from jax.experimental.pallas import tpu_sc as plsc

B, H, W, C = 4, 224, 224, 192
CP = 256
HW = H * W
NC, NS, L = 2, 16, 16
NW = NC * NS
PPW = HW // NW
CH = 32
NCHUNK = PPW // CH


def _idx_kernel(fi_ref, fj_ref, r0_ref, wi_ref, wj_ref):
    gi = lax.broadcasted_iota(jnp.int32, (1, H, W), 1).astype(jnp.float32)
    gj = lax.broadcasted_iota(jnp.int32, (1, H, W), 2).astype(jnp.float32)
    ti = jnp.clip(gi + fi_ref[...], 0.0, H - 1.0)
    tj = jnp.clip(gj + fj_ref[...], 0.0, W - 1.0)
    is_ = jnp.minimum(jnp.floor(ti), H - 2.0)
    js_ = jnp.minimum(jnp.floor(tj), W - 2.0)
    wi_ref[...] = ti - is_
    wj_ref[...] = tj - js_
    ii = is_.astype(jnp.int32)
    jj = js_.astype(jnp.int32)
    r0_ref[...] = ii * W + jj


def _indices_and_weights(flow):
    fi = flow[..., 0]
    fj = flow[..., 1]
    spec = pl.BlockSpec((1, H, W), lambda b: (b, 0, 0))
    r0, wi, wj = pl.pallas_call(
        _idx_kernel,
        grid=(B,),
        in_specs=[spec, spec],
        out_specs=[spec, spec, spec],
        out_shape=[
            jax.ShapeDtypeStruct((B, H, W), jnp.int32),
            jax.ShapeDtypeStruct((B, H, W), jnp.float32),
            jax.ShapeDtypeStruct((B, H, W), jnp.float32),
        ],
    )(fi, fj)
    return r0.reshape(B, HW), wi.reshape(B, HW), wj.reshape(B, HW)


_TR = 16


def _pad_kernel(v_ref, o_ref):
    for r in range(_TR):
        o_ref[pl.ds(r * W, W), :C] = v_ref[r].T
    o_ref[:, C:] = jnp.zeros((_TR * W, CP - C), jnp.float32)


def _padded_table(volb):
    volt = volb.transpose(0, 2, 1)
    return pl.pallas_call(
        _pad_kernel,
        grid=(H // _TR,),
        in_specs=[pl.BlockSpec((_TR, C, W), lambda g: (g, 0, 0))],
        out_specs=pl.BlockSpec((_TR * W, CP), lambda g: (g, 0)),
        out_shape=jax.ShapeDtypeStruct((HW, CP), jnp.float32),
    )(volt)


def _untranspose_kernel(v_ref, o_ref):
    for r in range(_TR):
        o_ref[r] = v_ref[pl.ds(r * W, W), :].T


def _untranspose(outc):
    return pl.pallas_call(
        _untranspose_kernel,
        grid=(H // _TR,),
        in_specs=[pl.BlockSpec((_TR * W, C), lambda g: (g, 0))],
        out_specs=pl.BlockSpec((_TR, C, W), lambda g: (g, 0, 0)),
        out_shape=jax.ShapeDtypeStruct((H, C, W), jnp.float32),
    )(outc)


def _sc_warp_body(tab_hbm, r0_hbm, fi_hbm, fj_hbm, out_hbm,
                  r0_v, fi_v, fj_v, idxA, idxB, bufA, bufB, outb,
                  semA, semB):
    wid = lax.axis_index("s") * NC + lax.axis_index("c")
    base = wid * PPW

    pltpu.sync_copy(r0_hbm.at[pl.ds(base, PPW)], r0_v)
    pltpu.sync_copy(fi_hbm.at[pl.ds(base, PPW)], fi_v)
    pltpu.sync_copy(fj_hbm.at[pl.ds(base, PPW)], fj_v)

    def start_chunk(g, slot):
        for t in range(CH // L):
            v = r0_v[pl.ds(g * CH + t * L, L)]
            idxA[slot, pl.ds(t * L, L)] = v
            idxA[slot, pl.ds(CH + t * L, L)] = v + 1
            idxB[slot, pl.ds(t * L, L)] = v + W
            idxB[slot, pl.ds(CH + t * L, L)] = v + (W + 1)
        pltpu.make_async_copy(tab_hbm.at[idxA.at[slot]], bufA.at[slot], semA).start()
        pltpu.make_async_copy(tab_hbm.at[idxB.at[slot]], bufB.at[slot], semB).start()

    def finish_chunk(g, slot):
        pltpu.make_async_copy(tab_hbm.at[idxA.at[slot]], bufA.at[slot], semA).wait()
        pltpu.make_async_copy(tab_hbm.at[idxB.at[slot]], bufB.at[slot], semB).wait()

        def px(k):
            kk = jnp.full((L,), g * CH + k, jnp.int32)
            fiv = plsc.load_gather(fi_v, [kk])
            fjv = plsc.load_gather(fj_v, [kk])
            w00 = (1.0 - fiv) * (1.0 - fjv)
            w01 = (1.0 - fiv) * fjv
            w10 = fiv * (1.0 - fjv)
            w11 = fiv * fjv
            for v in range(C // L):
                sl = pl.ds(v * L, L)
                o = (w00 * bufA[slot, k, sl] + w01 * bufA[slot, CH + k, sl]
                     + w10 * bufB[slot, k, sl] + w11 * bufB[slot, CH + k, sl])
                outb[slot, k, sl] = o

        plsc.parallel_loop(0, CH, 1, unroll=4)(px)
        pltpu.sync_copy(outb.at[slot], out_hbm.at[pl.ds(base + g * CH, CH)])

    start_chunk(0, 0)

    def chunk_body(g, _):
        slot = lax.rem(g, 2)
        nslot = 1 - slot

        @pl.when(g + 1 < NCHUNK)
        def _():
            start_chunk(g + 1, nslot)

        finish_chunk(g, slot)
        return 0

    lax.fori_loop(0, NCHUNK, chunk_body, 0)


def _sc_warp(tab, r0, fi, fj):
    mesh = plsc.VectorSubcoreMesh(core_axis_name="c", subcore_axis_name="s",
                                  num_cores=NC, num_subcores=NS)
    f = pl.kernel(
        _sc_warp_body,
        out_type=jax.ShapeDtypeStruct((HW, C), jnp.float32),
        mesh=mesh,
        compiler_params=pltpu.CompilerParams(needs_layout_passes=False),
        scratch_types=[
            pltpu.VMEM((PPW,), jnp.int32),
            pltpu.VMEM((PPW,), jnp.float32),
            pltpu.VMEM((PPW,), jnp.float32),
            pltpu.VMEM((2, 2 * CH), jnp.int32),
            pltpu.VMEM((2, 2 * CH), jnp.int32),
            pltpu.VMEM((2, 2 * CH, CP), jnp.float32),
            pltpu.VMEM((2, 2 * CH, CP), jnp.float32),
            pltpu.VMEM((2, CH, C), jnp.float32),
            pltpu.SemaphoreType.DMA,
            pltpu.SemaphoreType.DMA,
        ],
    )
    return f(tab, r0, fi, fj)


def kernel(vol, flow):
    r0, fi, fj = _indices_and_weights(flow)
    outs = []
    for b in range(B):
        tab = _padded_table(vol[b])
        outc = _sc_warp(tab, r0[b], fi[b], fj[b])
        outs.append(_untranspose(outc))
    return jnp.stack(outs).transpose(0, 1, 3, 2)


# speedup vs baseline: 1.3425x; 1.3425x over previous
import jax
import jax.numpy as jnp
from jax import lax
from jax.experimental import pallas as pl
from jax.experimental.pallas import tpu as pltpu
from jax.experimental.pallas import tpu_sc as plsc

B, H, W, C = 4, 224, 224, 192
CP = 256
NP = B * H * W
NC, NS, L = 2, 16, 16
NW = NC * NS
PPW = NP // NW
CH = 32
NCHUNK = PPW // CH


def _idx_kernel(fi_ref, fj_ref, r0_ref, wi_ref, wj_ref):
    b = pl.program_id(0)
    gi = lax.broadcasted_iota(jnp.int32, (1, H, W), 1).astype(jnp.float32)
    gj = lax.broadcasted_iota(jnp.int32, (1, H, W), 2).astype(jnp.float32)
    ti = jnp.clip(gi + fi_ref[...], 0.0, H - 1.0)
    tj = jnp.clip(gj + fj_ref[...], 0.0, W - 1.0)
    is_ = jnp.minimum(jnp.floor(ti), H - 2.0)
    js_ = jnp.minimum(jnp.floor(tj), W - 2.0)
    wi_ref[...] = ti - is_
    wj_ref[...] = tj - js_
    ii = is_.astype(jnp.int32)
    jj = js_.astype(jnp.int32)
    r0_ref[...] = (b * H + ii) * W + jj


def _indices_and_weights(flow):
    fi = flow[..., 0]
    fj = flow[..., 1]
    spec = pl.BlockSpec((1, H, W), lambda b: (b, 0, 0))
    r0, wi, wj = pl.pallas_call(
        _idx_kernel,
        grid=(B,),
        in_specs=[spec, spec],
        out_specs=[spec, spec, spec],
        out_shape=[
            jax.ShapeDtypeStruct((B, H, W), jnp.int32),
            jax.ShapeDtypeStruct((B, H, W), jnp.float32),
            jax.ShapeDtypeStruct((B, H, W), jnp.float32),
        ],
    )(fi, fj)
    return r0.reshape(NP), wi.reshape(NP), wj.reshape(NP)


_TR = 16


def _pad_kernel(v_ref, o_ref):
    for r in range(_TR):
        o_ref[pl.ds(r * W, W), :C] = v_ref[r].T
    o_ref[:, C:] = jnp.zeros((_TR * W, CP - C), jnp.float32)


def _padded_table(vol):
    volt = vol.transpose(0, 1, 3, 2).reshape(B * H, C, W)
    return pl.pallas_call(
        _pad_kernel,
        grid=(B * H // _TR,),
        in_specs=[pl.BlockSpec((_TR, C, W), lambda g: (g, 0, 0))],
        out_specs=pl.BlockSpec((_TR * W, CP), lambda g: (g, 0)),
        out_shape=jax.ShapeDtypeStruct((NP, CP), jnp.float32),
    )(volt)


def _untranspose_kernel(v_ref, o_ref):
    for r in range(_TR):
        o_ref[r] = v_ref[pl.ds(r * W, W), :].T


def _untranspose(outc):
    res = pl.pallas_call(
        _untranspose_kernel,
        grid=(B * H // _TR,),
        in_specs=[pl.BlockSpec((_TR * W, C), lambda g: (g, 0))],
        out_specs=pl.BlockSpec((_TR, C, W), lambda g: (g, 0, 0)),
        out_shape=jax.ShapeDtypeStruct((B * H, C, W), jnp.float32),
    )(outc)
    return res.reshape(B, H, C, W).transpose(0, 1, 3, 2)


def _sc_warp_body(tab_hbm, r0_hbm, fi_hbm, fj_hbm, out_hbm,
                  r0_v, fi_v, fj_v, idxA, idxB, bufA, bufB, outb,
                  semA, semB, semO):
    wid = lax.axis_index("s") * NC + lax.axis_index("c")
    base = wid * PPW

    pltpu.sync_copy(r0_hbm.at[pl.ds(base, PPW)], r0_v)
    pltpu.sync_copy(fi_hbm.at[pl.ds(base, PPW)], fi_v)
    pltpu.sync_copy(fj_hbm.at[pl.ds(base, PPW)], fj_v)

    def out_copy(g, slot):
        return pltpu.make_async_copy(
            outb.at[slot], out_hbm.at[pl.ds(base + g * CH, CH)], semO)

    def start_chunk(g, slot):
        for t in range(CH // L):
            v = r0_v[pl.ds(g * CH + t * L, L)]
            idxA[slot, pl.ds(t * L, L)] = v
            idxA[slot, pl.ds(CH + t * L, L)] = v + 1
            idxB[slot, pl.ds(t * L, L)] = v + W
            idxB[slot, pl.ds(CH + t * L, L)] = v + (W + 1)
        pltpu.make_async_copy(tab_hbm.at[idxA.at[slot]], bufA.at[slot], semA).start()
        pltpu.make_async_copy(tab_hbm.at[idxB.at[slot]], bufB.at[slot], semB).start()

    def finish_chunk(g, slot):
        pltpu.make_async_copy(tab_hbm.at[idxA.at[slot]], bufA.at[slot], semA).wait()
        pltpu.make_async_copy(tab_hbm.at[idxB.at[slot]], bufB.at[slot], semB).wait()

        def px(k):
            kk = jnp.full((L,), g * CH + k, jnp.int32)
            fiv = plsc.load_gather(fi_v, [kk])
            fjv = plsc.load_gather(fj_v, [kk])
            w00 = (1.0 - fiv) * (1.0 - fjv)
            w01 = (1.0 - fiv) * fjv
            w10 = fiv * (1.0 - fjv)
            w11 = fiv * fjv
            for v in range(C // L):
                sl = pl.ds(v * L, L)
                o = (w00 * bufA[slot, k, sl] + w01 * bufA[slot, CH + k, sl]
                     + w10 * bufB[slot, k, sl] + w11 * bufB[slot, CH + k, sl])
                outb[slot, k, sl] = o

        plsc.parallel_loop(0, CH, 1, unroll=4)(px)
        out_copy(g, slot).start()

    start_chunk(0, 0)

    def chunk_body(g, _):
        slot = lax.rem(g, 2)
        nslot = 1 - slot

        @pl.when(g + 1 < NCHUNK)
        def _():
            start_chunk(g + 1, nslot)

        @pl.when(g >= 2)
        def _():
            out_copy(g - 2, slot).wait()

        finish_chunk(g, slot)
        return 0

    lax.fori_loop(0, NCHUNK, chunk_body, 0)
    out_copy(NCHUNK - 2, lax.rem(NCHUNK - 2, 2)).wait()
    out_copy(NCHUNK - 1, lax.rem(NCHUNK - 1, 2)).wait()


def _sc_warp(tab, r0, fi, fj):
    mesh = plsc.VectorSubcoreMesh(core_axis_name="c", subcore_axis_name="s",
                                  num_cores=NC, num_subcores=NS)
    f = pl.kernel(
        _sc_warp_body,
        out_type=jax.ShapeDtypeStruct((NP, C), jnp.float32),
        mesh=mesh,
        compiler_params=pltpu.CompilerParams(needs_layout_passes=False),
        scratch_types=[
            pltpu.VMEM((PPW,), jnp.int32),
            pltpu.VMEM((PPW,), jnp.float32),
            pltpu.VMEM((PPW,), jnp.float32),
            pltpu.VMEM((2, 2 * CH), jnp.int32),
            pltpu.VMEM((2, 2 * CH), jnp.int32),
            pltpu.VMEM((2, 2 * CH, CP), jnp.float32),
            pltpu.VMEM((2, 2 * CH, CP), jnp.float32),
            pltpu.VMEM((2, CH, C), jnp.float32),
            pltpu.SemaphoreType.DMA,
            pltpu.SemaphoreType.DMA,
            pltpu.SemaphoreType.DMA,
        ],
    )
    return f(tab, r0, fi, fj)


def kernel(vol, flow):
    r0, fi, fj = _indices_and_weights(flow)
    tab = _padded_table(vol)
    out = _sc_warp(tab, r0, fi, fj)
    return _untranspose(out)


# speedup vs baseline: 1.3475x; 1.0037x over previous
import jax
import jax.numpy as jnp
from jax import lax
from jax.experimental import pallas as pl
from jax.experimental.pallas import tpu as pltpu
from jax.experimental.pallas import tpu_sc as plsc

B, H, W, C = 4, 224, 224, 192
CP = 256
NP = B * H * W
NC, NS, L = 2, 16, 16
NW = NC * NS
PPW = NP // NW
CH = 32
NCHUNK = PPW // CH


def _idx_kernel(fi_ref, fj_ref, r0_ref, wi_ref, wj_ref):
    b = pl.program_id(0)
    gi = lax.broadcasted_iota(jnp.int32, (1, H, W), 1).astype(jnp.float32)
    gj = lax.broadcasted_iota(jnp.int32, (1, H, W), 2).astype(jnp.float32)
    ti = jnp.clip(gi + fi_ref[...], 0.0, H - 1.0)
    tj = jnp.clip(gj + fj_ref[...], 0.0, W - 1.0)
    is_ = jnp.minimum(jnp.floor(ti), H - 2.0)
    js_ = jnp.minimum(jnp.floor(tj), W - 2.0)
    wi_ref[...] = ti - is_
    wj_ref[...] = tj - js_
    ii = is_.astype(jnp.int32)
    jj = js_.astype(jnp.int32)
    r0_ref[...] = (b * H + ii) * W + jj


def _indices_and_weights(flow):
    fi = flow[..., 0]
    fj = flow[..., 1]
    spec = pl.BlockSpec((1, H, W), lambda b: (b, 0, 0))
    r0, wi, wj = pl.pallas_call(
        _idx_kernel,
        grid=(B,),
        in_specs=[spec, spec],
        out_specs=[spec, spec, spec],
        out_shape=[
            jax.ShapeDtypeStruct((B, H, W), jnp.int32),
            jax.ShapeDtypeStruct((B, H, W), jnp.float32),
            jax.ShapeDtypeStruct((B, H, W), jnp.float32),
        ],
    )(fi, fj)
    return r0.reshape(NP), wi.reshape(NP), wj.reshape(NP)


_TR = 16


def _pad_kernel(v_ref, o_ref):
    for r in range(_TR):
        o_ref[pl.ds(r * W, W), :C] = v_ref[r].T
    o_ref[:, C:] = jnp.zeros((_TR * W, CP - C), jnp.float32)


def _padded_table(vol):
    volt = vol.transpose(0, 1, 3, 2).reshape(B * H, C, W)
    return pl.pallas_call(
        _pad_kernel,
        grid=(B * H // _TR,),
        in_specs=[pl.BlockSpec((_TR, C, W), lambda g: (g, 0, 0))],
        out_specs=pl.BlockSpec((_TR * W, CP), lambda g: (g, 0)),
        out_shape=jax.ShapeDtypeStruct((NP, CP), jnp.float32),
    )(volt)


def _untranspose_kernel(v_ref, o_ref):
    for r in range(_TR):
        o_ref[r] = v_ref[pl.ds(r * W, W), :].T


def _untranspose(outc):
    res = pl.pallas_call(
        _untranspose_kernel,
        grid=(B * H // _TR,),
        in_specs=[pl.BlockSpec((_TR * W, C), lambda g: (g, 0))],
        out_specs=pl.BlockSpec((_TR, C, W), lambda g: (g, 0, 0)),
        out_shape=jax.ShapeDtypeStruct((B * H, C, W), jnp.float32),
    )(outc)
    return res.reshape(B, H, C, W).transpose(0, 1, 3, 2)


def _sc_warp_body(tab_hbm, r0_hbm, fi_hbm, fj_hbm, out_hbm,
                  r0_v, fi_v, fj_v, idxAB, bufAB, outb,
                  semA, semO):
    wid = lax.axis_index("s") * NC + lax.axis_index("c")
    base = wid * PPW

    pltpu.sync_copy(r0_hbm.at[pl.ds(base, PPW)], r0_v)
    pltpu.sync_copy(fi_hbm.at[pl.ds(base, PPW)], fi_v)
    pltpu.sync_copy(fj_hbm.at[pl.ds(base, PPW)], fj_v)

    def out_copy(g, slot):
        return pltpu.make_async_copy(
            outb.at[slot], out_hbm.at[pl.ds(base + g * CH, CH)], semO)

    def start_chunk(g, slot):
        for t in range(CH // L):
            v = r0_v[pl.ds(g * CH + t * L, L)]
            idxAB[slot, pl.ds(t * L, L)] = v
            idxAB[slot, pl.ds(CH + t * L, L)] = v + 1
            idxAB[slot, pl.ds(2 * CH + t * L, L)] = v + W
            idxAB[slot, pl.ds(3 * CH + t * L, L)] = v + (W + 1)
        pltpu.make_async_copy(tab_hbm.at[idxAB.at[slot]], bufAB.at[slot], semA).start()

    def finish_chunk(g, slot):
        pltpu.make_async_copy(tab_hbm.at[idxAB.at[slot]], bufAB.at[slot], semA).wait()

        def px(k):
            kk = jnp.full((L,), g * CH + k, jnp.int32)
            fiv = plsc.load_gather(fi_v, [kk])
            fjv = plsc.load_gather(fj_v, [kk])
            w00 = (1.0 - fiv) * (1.0 - fjv)
            w01 = (1.0 - fiv) * fjv
            w10 = fiv * (1.0 - fjv)
            w11 = fiv * fjv
            for v in range(C // L):
                sl = pl.ds(v * L, L)
                o = (w00 * bufAB[slot, k, sl] + w01 * bufAB[slot, CH + k, sl]
                     + w10 * bufAB[slot, 2 * CH + k, sl]
                     + w11 * bufAB[slot, 3 * CH + k, sl])
                outb[slot, k, sl] = o

        plsc.parallel_loop(0, CH, 1, unroll=4)(px)
        out_copy(g, slot).start()

    start_chunk(0, 0)

    def chunk_body(g, _):
        slot = lax.rem(g, 2)
        nslot = 1 - slot

        @pl.when(g + 1 < NCHUNK)
        def _():
            start_chunk(g + 1, nslot)

        @pl.when(g >= 2)
        def _():
            out_copy(g - 2, slot).wait()

        finish_chunk(g, slot)
        return 0

    lax.fori_loop(0, NCHUNK, chunk_body, 0)
    out_copy(NCHUNK - 2, lax.rem(NCHUNK - 2, 2)).wait()
    out_copy(NCHUNK - 1, lax.rem(NCHUNK - 1, 2)).wait()


def _sc_warp(tab, r0, fi, fj):
    mesh = plsc.VectorSubcoreMesh(core_axis_name="c", subcore_axis_name="s",
                                  num_cores=NC, num_subcores=NS)
    f = pl.kernel(
        _sc_warp_body,
        out_type=jax.ShapeDtypeStruct((NP, C), jnp.float32),
        mesh=mesh,
        compiler_params=pltpu.CompilerParams(needs_layout_passes=False),
        scratch_types=[
            pltpu.VMEM((PPW,), jnp.int32),
            pltpu.VMEM((PPW,), jnp.float32),
            pltpu.VMEM((PPW,), jnp.float32),
            pltpu.VMEM((2, 4 * CH), jnp.int32),
            pltpu.VMEM((2, 4 * CH, CP), jnp.float32),
            pltpu.VMEM((2, CH, C), jnp.float32),
            pltpu.SemaphoreType.DMA,
            pltpu.SemaphoreType.DMA,
        ],
    )
    return f(tab, r0, fi, fj)


def kernel(vol, flow):
    r0, fi, fj = _indices_and_weights(flow)
    tab = _padded_table(vol)
    out = _sc_warp(tab, r0, fi, fj)
    return _untranspose(out)


# speedup vs baseline: 1.3775x; 1.0223x over previous
import jax
import jax.numpy as jnp
from jax import lax
from jax.experimental import pallas as pl
from jax.experimental.pallas import tpu as pltpu
from jax.experimental.pallas import tpu_sc as plsc

B, H, W, C = 4, 224, 224, 192
CP = 256
NP = B * H * W
NC, NS, L = 2, 16, 16
NW = NC * NS
PPW = NP // NW
CH = 32
NCHUNK = PPW // CH


def _idx_kernel(fi_ref, fj_ref, r0_ref, wi_ref, wj_ref):
    b = pl.program_id(0)
    gi = lax.broadcasted_iota(jnp.int32, (1, H, W), 1).astype(jnp.float32)
    gj = lax.broadcasted_iota(jnp.int32, (1, H, W), 2).astype(jnp.float32)
    ti = jnp.clip(gi + fi_ref[...], 0.0, H - 1.0)
    tj = jnp.clip(gj + fj_ref[...], 0.0, W - 1.0)
    is_ = jnp.minimum(jnp.floor(ti), H - 2.0)
    js_ = jnp.minimum(jnp.floor(tj), W - 2.0)
    wi_ref[...] = ti - is_
    wj_ref[...] = tj - js_
    ii = is_.astype(jnp.int32)
    jj = js_.astype(jnp.int32)
    r0_ref[...] = (b * H + ii) * W + jj


def _indices_and_weights(flow):
    fi = flow[..., 0]
    fj = flow[..., 1]
    spec = pl.BlockSpec((1, H, W), lambda b: (b, 0, 0))
    r0, wi, wj = pl.pallas_call(
        _idx_kernel,
        grid=(B,),
        in_specs=[spec, spec],
        out_specs=[spec, spec, spec],
        out_shape=[
            jax.ShapeDtypeStruct((B, H, W), jnp.int32),
            jax.ShapeDtypeStruct((B, H, W), jnp.float32),
            jax.ShapeDtypeStruct((B, H, W), jnp.float32),
        ],
    )(fi, fj)
    return r0.reshape(NP), wi.reshape(NP), wj.reshape(NP)


_TR = 32


def _pad_kernel(v_ref, o_ref):
    for r in range(_TR):
        o_ref[pl.ds(r * W, W), :C] = v_ref[r].T
    o_ref[:, C:] = jnp.zeros((_TR * W, CP - C), jnp.float32)


def _padded_table(vol):
    volt = vol.transpose(0, 1, 3, 2).reshape(B * H, C, W)
    return pl.pallas_call(
        _pad_kernel,
        grid=(B * H // _TR,),
        in_specs=[pl.BlockSpec((_TR, C, W), lambda g: (g, 0, 0))],
        out_specs=pl.BlockSpec((_TR * W, CP), lambda g: (g, 0)),
        out_shape=jax.ShapeDtypeStruct((NP, CP), jnp.float32),
    )(volt)


def _untranspose_kernel(v_ref, o_ref):
    for r in range(_TR):
        o_ref[r] = v_ref[pl.ds(r * W, W), :].T


def _untranspose(outc):
    res = pl.pallas_call(
        _untranspose_kernel,
        grid=(B * H // _TR,),
        in_specs=[pl.BlockSpec((_TR * W, C), lambda g: (g, 0))],
        out_specs=pl.BlockSpec((_TR, C, W), lambda g: (g, 0, 0)),
        out_shape=jax.ShapeDtypeStruct((B * H, C, W), jnp.float32),
    )(outc)
    return res.reshape(B, H, C, W).transpose(0, 1, 3, 2)


def _sc_warp_body(tab_hbm, r0_hbm, fi_hbm, fj_hbm, out_hbm,
                  r0_v, fi_v, fj_v, idxAB, bufAB, outb,
                  semA, semO):
    wid = lax.axis_index("s") * NC + lax.axis_index("c")
    base = wid * PPW

    pltpu.sync_copy(r0_hbm.at[pl.ds(base, PPW)], r0_v)
    pltpu.sync_copy(fi_hbm.at[pl.ds(base, PPW)], fi_v)
    pltpu.sync_copy(fj_hbm.at[pl.ds(base, PPW)], fj_v)

    def out_copy(g, slot):
        return pltpu.make_async_copy(
            outb.at[slot], out_hbm.at[pl.ds(base + g * CH, CH)], semO)

    def start_chunk(g, slot):
        for t in range(CH // L):
            v = r0_v[pl.ds(g * CH + t * L, L)]
            idxAB[slot, pl.ds(t * L, L)] = v
            idxAB[slot, pl.ds(CH + t * L, L)] = v + 1
            idxAB[slot, pl.ds(2 * CH + t * L, L)] = v + W
            idxAB[slot, pl.ds(3 * CH + t * L, L)] = v + (W + 1)
        pltpu.make_async_copy(tab_hbm.at[idxAB.at[slot]], bufAB.at[slot], semA).start()

    def finish_chunk(g, slot):
        pltpu.make_async_copy(tab_hbm.at[idxAB.at[slot]], bufAB.at[slot], semA).wait()

        def px(k):
            kk = jnp.full((L,), g * CH + k, jnp.int32)
            fiv = plsc.load_gather(fi_v, [kk])
            fjv = plsc.load_gather(fj_v, [kk])
            w00 = (1.0 - fiv) * (1.0 - fjv)
            w01 = (1.0 - fiv) * fjv
            w10 = fiv * (1.0 - fjv)
            w11 = fiv * fjv
            for v in range(C // L):
                sl = pl.ds(v * L, L)
                o = (w00 * bufAB[slot, k, sl] + w01 * bufAB[slot, CH + k, sl]
                     + w10 * bufAB[slot, 2 * CH + k, sl]
                     + w11 * bufAB[slot, 3 * CH + k, sl])
                outb[slot, k, sl] = o

        plsc.parallel_loop(0, CH, 1, unroll=4)(px)
        out_copy(g, slot).start()

    start_chunk(0, 0)

    def chunk_body(g, _):
        slot = lax.rem(g, 2)
        nslot = 1 - slot

        @pl.when(g + 1 < NCHUNK)
        def _():
            start_chunk(g + 1, nslot)

        @pl.when(g >= 2)
        def _():
            out_copy(g - 2, slot).wait()

        finish_chunk(g, slot)
        return 0

    lax.fori_loop(0, NCHUNK, chunk_body, 0)
    out_copy(NCHUNK - 2, lax.rem(NCHUNK - 2, 2)).wait()
    out_copy(NCHUNK - 1, lax.rem(NCHUNK - 1, 2)).wait()


def _sc_warp(tab, r0, fi, fj):
    mesh = plsc.VectorSubcoreMesh(core_axis_name="c", subcore_axis_name="s",
                                  num_cores=NC, num_subcores=NS)
    f = pl.kernel(
        _sc_warp_body,
        out_type=jax.ShapeDtypeStruct((NP, C), jnp.float32),
        mesh=mesh,
        compiler_params=pltpu.CompilerParams(needs_layout_passes=False),
        scratch_types=[
            pltpu.VMEM((PPW,), jnp.int32),
            pltpu.VMEM((PPW,), jnp.float32),
            pltpu.VMEM((PPW,), jnp.float32),
            pltpu.VMEM((2, 4 * CH), jnp.int32),
            pltpu.VMEM((2, 4 * CH, CP), jnp.float32),
            pltpu.VMEM((2, CH, C), jnp.float32),
            pltpu.SemaphoreType.DMA,
            pltpu.SemaphoreType.DMA,
        ],
    )
    return f(tab, r0, fi, fj)


def kernel(vol, flow):
    r0, fi, fj = _indices_and_weights(flow)
    tab = _padded_table(vol)
    out = _sc_warp(tab, r0, fi, fj)
    return _untranspose(out)


# speedup vs baseline: 1.3790x; 1.0011x over previous
import jax
import jax.numpy as jnp
from jax import lax
from jax.experimental import pallas as pl
from jax.experimental.pallas import tpu as pltpu
from jax.experimental.pallas import tpu_sc as plsc

B, H, W, C = 4, 224, 224, 192
CP = 256
NP = B * H * W
NC, NS, L = 2, 16, 16
NW = NC * NS
PPW = NP // NW
CH = 32
NCHUNK = PPW // CH


def _idx_kernel(fi_ref, fj_ref, r0_ref, wi_ref, wj_ref):
    b = pl.program_id(0)
    gi = lax.broadcasted_iota(jnp.int32, (1, H, W), 1).astype(jnp.float32)
    gj = lax.broadcasted_iota(jnp.int32, (1, H, W), 2).astype(jnp.float32)
    ti = jnp.clip(gi + fi_ref[...], 0.0, H - 1.0)
    tj = jnp.clip(gj + fj_ref[...], 0.0, W - 1.0)
    is_ = jnp.minimum(jnp.floor(ti), H - 2.0)
    js_ = jnp.minimum(jnp.floor(tj), W - 2.0)
    wi_ref[...] = ti - is_
    wj_ref[...] = tj - js_
    ii = is_.astype(jnp.int32)
    jj = js_.astype(jnp.int32)
    r0_ref[...] = (b * H + ii) * W + jj


def _indices_and_weights(flow):
    fi = flow[..., 0]
    fj = flow[..., 1]
    spec = pl.BlockSpec((1, H, W), lambda b: (b, 0, 0))
    r0, wi, wj = pl.pallas_call(
        _idx_kernel,
        grid=(B,),
        in_specs=[spec, spec],
        out_specs=[spec, spec, spec],
        out_shape=[
            jax.ShapeDtypeStruct((B, H, W), jnp.int32),
            jax.ShapeDtypeStruct((B, H, W), jnp.float32),
            jax.ShapeDtypeStruct((B, H, W), jnp.float32),
        ],
    )(fi, fj)
    return r0.reshape(NP), wi.reshape(NP), wj.reshape(NP)


_TR = 56


def _pad_kernel(v_ref, o_ref):
    for r in range(_TR):
        o_ref[pl.ds(r * W, W), :C] = v_ref[r].T
    o_ref[:, C:] = jnp.zeros((_TR * W, CP - C), jnp.float32)


def _padded_table(vol):
    volt = vol.transpose(0, 1, 3, 2).reshape(B * H, C, W)
    return pl.pallas_call(
        _pad_kernel,
        grid=(B * H // _TR,),
        in_specs=[pl.BlockSpec((_TR, C, W), lambda g: (g, 0, 0))],
        out_specs=pl.BlockSpec((_TR * W, CP), lambda g: (g, 0)),
        out_shape=jax.ShapeDtypeStruct((NP, CP), jnp.float32),
    )(volt)


def _untranspose_kernel(v_ref, o_ref):
    for r in range(_TR):
        o_ref[r] = v_ref[pl.ds(r * W, W), :].T


def _untranspose(outc):
    res = pl.pallas_call(
        _untranspose_kernel,
        grid=(B * H // _TR,),
        in_specs=[pl.BlockSpec((_TR * W, C), lambda g: (g, 0))],
        out_specs=pl.BlockSpec((_TR, C, W), lambda g: (g, 0, 0)),
        out_shape=jax.ShapeDtypeStruct((B * H, C, W), jnp.float32),
    )(outc)
    return res.reshape(B, H, C, W).transpose(0, 1, 3, 2)


def _sc_warp_body(tab_hbm, r0_hbm, fi_hbm, fj_hbm, out_hbm,
                  r0_v, fi_v, fj_v, idxAB, bufAB, outb,
                  semA, semO):
    wid = lax.axis_index("s") * NC + lax.axis_index("c")
    base = wid * PPW

    pltpu.sync_copy(r0_hbm.at[pl.ds(base, PPW)], r0_v)
    pltpu.sync_copy(fi_hbm.at[pl.ds(base, PPW)], fi_v)
    pltpu.sync_copy(fj_hbm.at[pl.ds(base, PPW)], fj_v)

    def out_copy(g, slot):
        return pltpu.make_async_copy(
            outb.at[slot], out_hbm.at[pl.ds(base + g * CH, CH)], semO)

    def start_chunk(g, slot):
        for t in range(CH // L):
            v = r0_v[pl.ds(g * CH + t * L, L)]
            idxAB[slot, pl.ds(t * L, L)] = v
            idxAB[slot, pl.ds(CH + t * L, L)] = v + 1
            idxAB[slot, pl.ds(2 * CH + t * L, L)] = v + W
            idxAB[slot, pl.ds(3 * CH + t * L, L)] = v + (W + 1)
        pltpu.make_async_copy(tab_hbm.at[idxAB.at[slot]], bufAB.at[slot], semA).start()

    def finish_chunk(g, slot):
        pltpu.make_async_copy(tab_hbm.at[idxAB.at[slot]], bufAB.at[slot], semA).wait()

        def px(k):
            kk = jnp.full((L,), g * CH + k, jnp.int32)
            fiv = plsc.load_gather(fi_v, [kk])
            fjv = plsc.load_gather(fj_v, [kk])
            w00 = (1.0 - fiv) * (1.0 - fjv)
            w01 = (1.0 - fiv) * fjv
            w10 = fiv * (1.0 - fjv)
            w11 = fiv * fjv
            for v in range(C // L):
                sl = pl.ds(v * L, L)
                o = (w00 * bufAB[slot, k, sl] + w01 * bufAB[slot, CH + k, sl]
                     + w10 * bufAB[slot, 2 * CH + k, sl]
                     + w11 * bufAB[slot, 3 * CH + k, sl])
                outb[slot, k, sl] = o

        plsc.parallel_loop(0, CH, 1, unroll=4)(px)
        out_copy(g, slot).start()

    start_chunk(0, 0)

    def chunk_body(g, _):
        slot = lax.rem(g, 2)
        nslot = 1 - slot

        @pl.when(g + 1 < NCHUNK)
        def _():
            start_chunk(g + 1, nslot)

        @pl.when(g >= 2)
        def _():
            out_copy(g - 2, slot).wait()

        finish_chunk(g, slot)
        return 0

    lax.fori_loop(0, NCHUNK, chunk_body, 0)
    out_copy(NCHUNK - 2, lax.rem(NCHUNK - 2, 2)).wait()
    out_copy(NCHUNK - 1, lax.rem(NCHUNK - 1, 2)).wait()


def _sc_warp(tab, r0, fi, fj):
    mesh = plsc.VectorSubcoreMesh(core_axis_name="c", subcore_axis_name="s",
                                  num_cores=NC, num_subcores=NS)
    f = pl.kernel(
        _sc_warp_body,
        out_type=jax.ShapeDtypeStruct((NP, C), jnp.float32),
        mesh=mesh,
        compiler_params=pltpu.CompilerParams(needs_layout_passes=False),
        scratch_types=[
            pltpu.VMEM((PPW,), jnp.int32),
            pltpu.VMEM((PPW,), jnp.float32),
            pltpu.VMEM((PPW,), jnp.float32),
            pltpu.VMEM((2, 4 * CH), jnp.int32),
            pltpu.VMEM((2, 4 * CH, CP), jnp.float32),
            pltpu.VMEM((2, CH, C), jnp.float32),
            pltpu.SemaphoreType.DMA,
            pltpu.SemaphoreType.DMA,
        ],
    )
    return f(tab, r0, fi, fj)


def kernel(vol, flow):
    r0, fi, fj = _indices_and_weights(flow)
    tab = _padded_table(vol)
    out = _sc_warp(tab, r0, fi, fj)
    return _untranspose(out)
